# Initial kernel scaffold; baseline (speedup 1.0000x reference)
import jax
import jax.numpy as jnp
from jax.experimental import pallas as pl


def kernel(x, Wr, br, W1, b1, W2, b2):
    raise NotImplementedError("write your pallas kernel here")


# speedup vs baseline: 1.7284x; 1.7284x over previous
import functools

import jax
import jax.numpy as jnp
from jax.experimental import pallas as pl
from jax.experimental.pallas import tpu as pltpu

T = 8192
D = 768
F = 128
E = 8
TB = 512


def _moe_kernel(x_ref, wr_ref, br_ref, w1_ref, b1_ref, w2_ref, b2_ref,
                out_ref, imp_ref, loss_ref, *, num_tiles):
    i = pl.program_id(0)
    x = x_ref[...]

    logits = jnp.dot(x, wr_ref[...], preferred_element_type=jnp.float32)
    logits = logits + br_ref[...]
    m = jnp.max(logits, axis=-1, keepdims=True)
    ex = jnp.exp(logits - m)
    scores = ex / jnp.sum(ex, axis=-1, keepdims=True)

    iota = jax.lax.broadcasted_iota(jnp.int32, (TB, E), 1)
    v1 = jnp.max(scores, axis=-1, keepdims=True)
    idx1 = jnp.min(jnp.where(scores == v1, iota, E), axis=-1, keepdims=True)
    mask1 = iota == idx1
    s2 = jnp.where(mask1, -jnp.inf, scores)
    v2 = jnp.max(s2, axis=-1, keepdims=True)
    idx2 = jnp.min(jnp.where(s2 == v2, iota, E), axis=-1, keepdims=True)
    w = jnp.where(mask1 | (iota == idx2), scores, 0.0)

    @pl.when(i == 0)
    def _init():
        imp_ref[...] = jnp.zeros_like(imp_ref)

    imp_ref[...] += jnp.sum(w, axis=0).reshape(1, E)

    acc = jnp.zeros((TB, D), jnp.float32)
    for e_i in range(E):
        h = jnp.dot(x, w1_ref[e_i], preferred_element_type=jnp.float32)
        h = jnp.maximum(h + b1_ref[e_i], 0.0)
        o = jnp.dot(h, w2_ref[e_i], preferred_element_type=jnp.float32)
        o = o + b2_ref[e_i]
        acc = acc + w[:, e_i:e_i + 1] * o
    out_ref[...] = acc

    @pl.when(i == num_tiles - 1)
    def _loss():
        imp = imp_ref[0, :]
        mean = jnp.mean(imp)
        var = jnp.sum((imp - mean) ** 2) / (E - 1)
        loss_ref[...] = (var / (mean * mean + 1e-9)).reshape(1, 1)


def kernel(x, Wr, br, W1, b1, W2, b2):
    num_tiles = T // TB
    out, imp, loss = pl.pallas_call(
        functools.partial(_moe_kernel, num_tiles=num_tiles),
        grid=(num_tiles,),
        in_specs=[
            pl.BlockSpec((TB, D), lambda i: (i, 0)),
            pl.BlockSpec((D, E), lambda i: (0, 0)),
            pl.BlockSpec((1, E), lambda i: (0, 0)),
            pl.BlockSpec((E, D, F), lambda i: (0, 0, 0)),
            pl.BlockSpec((E, F), lambda i: (0, 0)),
            pl.BlockSpec((E, F, D), lambda i: (0, 0, 0)),
            pl.BlockSpec((E, D), lambda i: (0, 0)),
        ],
        out_specs=[
            pl.BlockSpec((TB, D), lambda i: (i, 0)),
            pl.BlockSpec((1, E), lambda i: (0, 0)),
            pl.BlockSpec((1, 1), lambda i: (0, 0)),
        ],
        out_shape=[
            jax.ShapeDtypeStruct((T, D), jnp.float32),
            jax.ShapeDtypeStruct((1, E), jnp.float32),
            jax.ShapeDtypeStruct((1, 1), jnp.float32),
        ],
        compiler_params=pltpu.CompilerParams(
            dimension_semantics=("arbitrary",),
        ),
    )(x, Wr, br.reshape(1, E), W1, b1, W2, b2)
    del imp
    return out, loss[0, 0]
